# baseline (speedup 1.0000x reference)
import jax
import jax.numpy as jnp
from jax.experimental import pallas as pl
from jax.experimental.pallas import tpu as pltpu

T = 2048
K = 16
QB = 256
SB = 128
SWB = SB + 2 * K
WB = QB + 2 * K
NBLK = T // QB
HIDDEN = 128
NHEAD_T = 4
DH = HIDDEN // NHEAD_T
FF = 512
NEG = -1e30

ENC1 = 0
ENC2 = ENC1 + NBLK
ENC3 = ENC2 + NBLK
QKV1 = ENC3 + NBLK
ATT1 = QKV1 + 1
QKV2 = ATT1 + NBLK
ATT2 = QKV2 + 1
CLS = ATT2 + NBLK
DEC1 = CLS + 1
DEC2 = DEC1 + NBLK
DEC3 = DEC2 + NBLK
NSTEP = DEC3 + NBLK

SCALE = 1.0 / (DH ** 0.5)


def _abt(a, b, precision=None):
    return jax.lax.dot_general(a, b, (((1,), (1,)), ((), ())),
                               preferred_element_type=jnp.float32,
                               precision=precision)


_HI = jax.lax.Precision.HIGHEST


def _gat_store(b, win, dst, w_ref, asrc_ref, adst_ref, b_ref, heads, outd,
               act):
    h_win = jnp.dot(win, w_ref[:, :], preferred_element_type=jnp.float32)
    iq = jax.lax.broadcasted_iota(jnp.int32, (QB, WB), 0)
    jk = jax.lax.broadcasted_iota(jnp.int32, (QB, WB), 1)
    g = b * QB + jk - K
    band = (jk - iq >= 0) & (jk - iq <= 2 * K) & (g >= 0) & (g < T)
    madd = jnp.where(band, 0.0, NEG)
    outs = []
    for hd in range(heads):
        hh = h_win[:, hd * outd:(hd + 1) * outd]
        a_src = _abt(asrc_ref[hd:hd + 1, :], hh, _HI)
        a_dst = _abt(hh[K:K + QB, :], adst_ref[hd:hd + 1, :], _HI)
        s = a_src + a_dst
        e = jnp.maximum(s, 0.2 * s) + madd
        m = jnp.max(e, axis=1, keepdims=True)
        w = jnp.exp(e - m)
        z = jnp.sum(w, axis=1, keepdims=True)
        o = jnp.dot(w, hh, preferred_element_type=jnp.float32)
        outs.append(o * (1.0 / (z + 1e-16)))
    out = jnp.concatenate(outs, axis=1) + b_ref[:]
    if act:
        out = jnp.maximum(out, 0.0)
    dst[pl.ds(K + b * QB, QB), :] = out


def _ln(x, w, b):
    mu = jnp.mean(x, axis=-1, keepdims=True)
    var = jnp.mean((x - mu) ** 2, axis=-1, keepdims=True)
    return (x - mu) * jax.lax.rsqrt(var + 1e-5) * w + b


def _att_compute(b, src, qkv_ref, wo_ref, bo_ref, l1w, l1b, w1, b1, w2, b2,
                 l2w, l2b):
    xb = src[pl.ds(K + b * QB, QB), :]
    outs = []
    for hd in range(NHEAD_T):
        qh = qkv_ref[pl.ds(b * QB, QB), hd * DH:(hd + 1) * DH] * SCALE
        kh = qkv_ref[:, HIDDEN + hd * DH:HIDDEN + (hd + 1) * DH]
        vh = qkv_ref[:, 2 * HIDDEN + hd * DH:2 * HIDDEN + (hd + 1) * DH]
        s = _abt(qh, kh)
        m = jnp.max(s, axis=1, keepdims=True)
        w = jnp.exp(s - m)
        z = jnp.sum(w, axis=1, keepdims=True)
        o = jnp.dot(w, vh, preferred_element_type=jnp.float32)
        outs.append(o * (1.0 / z))
    o = jnp.concatenate(outs, axis=1)
    o = _abt(o, wo_ref[:, :]) + bo_ref[:]
    x1 = _ln(xb + o, l1w[:], l1b[:])
    f = jnp.maximum(_abt(x1, w1[:, :]) + b1[:], 0.0)
    f = _abt(f, w2[:, :]) + b2[:]
    return _ln(x1 + f, l2w[:], l2b[:])


def _mega_body(xm_ref, x0_ref, xp_ref,
               w_e1, s_e1, d_e1, be1, w_e2, s_e2, d_e2, be2,
               w_e3, s_e3, d_e3, be3,
               wq1, bq1, wo1, bo1, aw1, ab1, f1a, f1ab, f1b, f1bb, g1w, g1b,
               wq2, bq2, wo2, bo2, aw2, ab2, f2a, f2ab, f2b, f2bb, g2w, g2b,
               clw1, clb1, clw2, clb2,
               w_d1, s_d1, d_d1, bd1, w_d2, s_d2, d_d2, bd2,
               wt_d3, s_d3, d_d3, bcol_d3,
               out_ref, logit_ref,
               bufA, bufB, qkvS, winS):
    i = pl.program_id(0)

    @pl.when(i == 0)
    def _():
        z = jnp.zeros((K, HIDDEN), jnp.float32)
        bufA[:K, :] = z
        bufA[K + T:, :] = z
        bufB[:K, :] = z
        bufB[K + T:, :] = z

    @pl.when(i < ENC2)
    def _():
        winS[:K, :] = xm_ref[QB - K:, :]
        winS[K:K + QB, :] = x0_ref[:, :]
        winS[K + QB:, :] = xp_ref[:K, :]
        _gat_store(i, winS[:, :], bufA, w_e1, s_e1, d_e1, be1, 4, 32, True)

    @pl.when((i >= ENC2) & (i < ENC3))
    def _():
        b = i - ENC2
        win = bufA[pl.ds(b * QB, WB), :]
        _gat_store(b, win, bufB, w_e2, s_e2, d_e2, be2, 4, 32, True)

    @pl.when((i >= ENC3) & (i < QKV1))
    def _():
        b = i - ENC3
        win = bufB[pl.ds(b * QB, WB), :]
        _gat_store(b, win, bufA, w_e3, s_e3, d_e3, be3, 4, 32, True)

    @pl.when(i == QKV1)
    def _():
        qkvS[:, :] = _abt(bufA[K:K + T, :], wq1[:, :]) + bq1[:]

    @pl.when((i >= ATT1) & (i < QKV2))
    def _():
        b = i - ATT1
        bufB[pl.ds(K + b * QB, QB), :] = _att_compute(
            b, bufA, qkvS, wo1, bo1, aw1, ab1, f1a, f1ab, f1b, f1bb,
            g1w, g1b)

    @pl.when(i == QKV2)
    def _():
        qkvS[:, :] = _abt(bufB[K:K + T, :], wq2[:, :]) + bq2[:]

    @pl.when((i >= ATT2) & (i < CLS))
    def _():
        b = i - ATT2
        bufA[pl.ds(K + b * QB, QB), :] = _att_compute(
            b, bufB, qkvS, wo2, bo2, aw2, ab2, f2a, f2ab, f2b, f2bb,
            g2w, g2b)

    @pl.when(i == CLS)
    def _():
        hc = jnp.mean(bufA[K:K + T, :], axis=0, keepdims=True)
        h1 = jnp.maximum(_abt(hc, clw1[:, :]) + clb1[:], 0.0)
        logit_ref[:, :] = _abt(h1, clw2[:, :]) + clb2[:]

    @pl.when((i >= DEC1) & (i < DEC2))
    def _():
        b = i - DEC1
        win = bufA[pl.ds(b * QB, WB), :]
        _gat_store(b, win, bufB, w_d1, s_d1, d_d1, bd1, 4, 32, True)

    @pl.when((i >= DEC2) & (i < DEC3))
    def _():
        b = i - DEC2
        win = bufB[pl.ds(b * QB, WB), :]
        _gat_store(b, win, bufA, w_d2, s_d2, d_d2, bd2, 4, 32, True)

    @pl.when(i >= DEC3)
    def _():
        b = i - DEC3
        win = bufA[pl.ds(b * QB, WB), :]
        hh = _abt(win, wt_d3[:, :], _HI)
        hhT = _abt(wt_d3[:, :], win, _HI)
        a_srcT = _abt(hh, s_d3[0:1, :], _HI)
        a_dstT = _abt(d_d3[0:1, :], hh[K:K + QB, :], _HI)
        jT = jax.lax.broadcasted_iota(jnp.int32, (WB, QB), 0)
        iqT = jax.lax.broadcasted_iota(jnp.int32, (WB, QB), 1)
        g = b * QB + jT - K
        band = (jT - iqT >= 0) & (jT - iqT <= 2 * K) & (g >= 0) & (g < T)
        madd = jnp.where(band, 0.0, NEG)
        s = a_srcT + a_dstT
        e = jnp.maximum(s, 0.2 * s) + madd
        m = jnp.max(e, axis=0, keepdims=True)
        w = jnp.exp(e - m)
        z = jnp.sum(w, axis=0, keepdims=True)
        oT = jnp.dot(hhT, w, preferred_element_type=jnp.float32,
                     precision=_HI)
        out_ref[:, :] = oT * (1.0 / (z + 1e-16)) + bcol_d3[:, :]


def kernel(window, params):
    x = window
    in_ch = x.shape[1]

    gats = []
    for p in params["enc"] + params["dec"][:-1]:
        gats += [p["W"], p["att_src"], p["att_dst"], p["b"]]
    t_ops = []
    for p in params["trans"]:
        t_ops += [p["Wqkv"], p["bqkv"], p["Wo"], p["bo"],
                  p["ln1_w"], p["ln1_b"], p["W1"], p["b1"],
                  p["W2"], p["b2"], p["ln2_w"], p["ln2_b"]]
    c = params["cls"]
    cls_ops = [c["W1"], c["b1"], c["W2"], c["b2"]]
    pd3 = params["dec"][-1]
    d3_ops = [pd3["W"].T, pd3["att_src"], pd3["att_dst"], pd3["b"][:, None]]

    operands = ([x, x, x] + gats[:12] + t_ops + cls_ops + gats[12:]
                + d3_ops)

    def _full(a):
        nd = a.ndim
        return pl.BlockSpec(a.shape, lambda i, _nd=nd: (0,) * _nd)

    in_specs = [
        pl.BlockSpec((QB, in_ch),
                     lambda i: (jnp.maximum(jnp.clip(i, 0, NBLK - 1) - 1, 0),
                                0)),
        pl.BlockSpec((QB, in_ch), lambda i: (jnp.clip(i, 0, NBLK - 1), 0)),
        pl.BlockSpec((QB, in_ch),
                     lambda i: (jnp.minimum(jnp.clip(i, 0, NBLK - 1) + 1,
                                            NBLK - 1), 0)),
    ] + [_full(a) for a in operands[3:]]

    out, logits = pl.pallas_call(
        _mega_body,
        grid=(NSTEP,),
        in_specs=in_specs,
        out_specs=[
            pl.BlockSpec((64, QB), lambda i: (0, jnp.clip(i - DEC3, 0,
                                                          NBLK - 1))),
            pl.BlockSpec((1, 2), lambda i: (0, 0)),
        ],
        out_shape=[
            jax.ShapeDtypeStruct((64, T), jnp.float32),
            jax.ShapeDtypeStruct((1, 2), jnp.float32),
        ],
        scratch_shapes=[
            pltpu.VMEM((T + 2 * K, HIDDEN), jnp.float32),
            pltpu.VMEM((T + 2 * K, HIDDEN), jnp.float32),
            pltpu.VMEM((T, 3 * HIDDEN), jnp.float32),
            pltpu.VMEM((WB, in_ch), jnp.float32),
        ],
    )(*operands)
    return (out, logits[0])


# speedup vs baseline: 1.1875x; 1.1875x over previous
import jax
import jax.numpy as jnp
from jax.experimental import pallas as pl
from jax.experimental.pallas import tpu as pltpu

T = 2048
K = 16
QB = 256
SB = 128
SWB = SB + 2 * K
WB = QB + 2 * K
NBLK = T // QB
HIDDEN = 128
NHEAD_T = 4
DH = HIDDEN // NHEAD_T
FF = 512
NEG = -1e30

ENC1 = 0
ENC2 = ENC1 + NBLK
ENC3 = ENC2 + NBLK
QKV1 = ENC3 + NBLK
ATT1 = QKV1 + 1
QKV2 = ATT1 + NBLK
ATT2 = QKV2 + 1
CLS = ATT2 + NBLK
DEC1 = CLS + 1
DEC2 = DEC1 + NBLK
DEC3 = DEC2 + NBLK
NSTEP = DEC3 + NBLK

SCALE = 1.0 / (DH ** 0.5)


def _abt(a, b):
    return jax.lax.dot_general(a, b, (((1,), (1,)), ((), ())),
                               preferred_element_type=jnp.float32)


def _gat_store(b, win, dst, w_ref, asrc_ref, adst_ref, b_ref, heads, outd,
               act):
    h_win = jnp.dot(win, w_ref[:, :], preferred_element_type=jnp.float32)
    a_src_all = _abt(asrc_ref[:, :], win)
    a_dst_all = jnp.dot(win[K:K + QB, :], adst_ref[:, :],
                        preferred_element_type=jnp.float32)
    iq = jax.lax.broadcasted_iota(jnp.int32, (QB, WB), 0)
    jk = jax.lax.broadcasted_iota(jnp.int32, (QB, WB), 1)
    g = b * QB + jk - K
    band = (jk - iq >= 0) & (jk - iq <= 2 * K) & (g >= 0) & (g < T)
    madd = jnp.where(band, 0.0, NEG)
    outs = []
    for hd in range(heads):
        hh = h_win[:, hd * outd:(hd + 1) * outd]
        s = a_src_all[hd:hd + 1, :] + a_dst_all[:, hd:hd + 1]
        e = jnp.maximum(s, 0.2 * s) + madd
        m = jnp.max(e, axis=1, keepdims=True)
        w = jnp.exp(e - m)
        z = jnp.sum(w, axis=1, keepdims=True)
        o = jnp.dot(w, hh, preferred_element_type=jnp.float32)
        outs.append(o * (1.0 / (z + 1e-16)))
    out = jnp.concatenate(outs, axis=1) + b_ref[:]
    if act:
        out = jnp.maximum(out, 0.0)
    dst[pl.ds(K + b * QB, QB), :] = out


def _ln(x, w, b):
    mu = jnp.mean(x, axis=-1, keepdims=True)
    var = jnp.mean((x - mu) ** 2, axis=-1, keepdims=True)
    return (x - mu) * jax.lax.rsqrt(var + 1e-5) * w + b


def _att_compute(b, src, qkv_ref, wo_ref, bo_ref, l1w, l1b, w1, b1, w2, b2,
                 l2w, l2b):
    xb = src[pl.ds(K + b * QB, QB), :]
    outs = []
    for hd in range(NHEAD_T):
        qh = qkv_ref[pl.ds(b * QB, QB), hd * DH:(hd + 1) * DH] * SCALE
        kh = qkv_ref[:, HIDDEN + hd * DH:HIDDEN + (hd + 1) * DH]
        vh = qkv_ref[:, 2 * HIDDEN + hd * DH:2 * HIDDEN + (hd + 1) * DH]
        s = _abt(qh, kh)
        m = jnp.max(s, axis=1, keepdims=True)
        w = jnp.exp(s - m)
        z = jnp.sum(w, axis=1, keepdims=True)
        o = jnp.dot(w, vh, preferred_element_type=jnp.float32)
        outs.append(o * (1.0 / z))
    o = jnp.concatenate(outs, axis=1)
    o = _abt(o, wo_ref[:, :]) + bo_ref[:]
    x1 = _ln(xb + o, l1w[:], l1b[:])
    f = jnp.maximum(_abt(x1, w1[:, :]) + b1[:], 0.0)
    f = _abt(f, w2[:, :]) + b2[:]
    return _ln(x1 + f, l2w[:], l2b[:])


def _mega_body(xm_ref, x0_ref, xp_ref,
               w_e1, s_e1, d_e1, be1, w_e2, s_e2, d_e2, be2,
               w_e3, s_e3, d_e3, be3,
               wq1, bq1, wo1, bo1, aw1, ab1, f1a, f1ab, f1b, f1bb, g1w, g1b,
               wq2, bq2, wo2, bo2, aw2, ab2, f2a, f2ab, f2b, f2bb, g2w, g2b,
               clw1, clb1, clw2, clb2,
               w_d1, s_d1, d_d1, bd1, w_d2, s_d2, d_d2, bd2,
               wt_d3, s_d3, d_d3, bcol_d3,
               out_ref, logit_ref,
               bufA, bufB, qkvS, winS):
    i = pl.program_id(0)

    @pl.when(i == 0)
    def _():
        z = jnp.zeros((K, HIDDEN), jnp.float32)
        bufA[:K, :] = z
        bufA[K + T:, :] = z
        bufB[:K, :] = z
        bufB[K + T:, :] = z

    @pl.when(i < ENC2)
    def _():
        winS[:K, :] = xm_ref[QB - K:, :]
        winS[K:K + QB, :] = x0_ref[:, :]
        winS[K + QB:, :] = xp_ref[:K, :]
        _gat_store(i, winS[:, :], bufA, w_e1, s_e1, d_e1, be1, 4, 32, True)

    @pl.when((i >= ENC2) & (i < ENC3))
    def _():
        b = i - ENC2
        win = bufA[pl.ds(b * QB, WB), :]
        _gat_store(b, win, bufB, w_e2, s_e2, d_e2, be2, 4, 32, True)

    @pl.when((i >= ENC3) & (i < QKV1))
    def _():
        b = i - ENC3
        win = bufB[pl.ds(b * QB, WB), :]
        _gat_store(b, win, bufA, w_e3, s_e3, d_e3, be3, 4, 32, True)

    @pl.when(i == QKV1)
    def _():
        qkvS[:, :] = _abt(bufA[K:K + T, :], wq1[:, :]) + bq1[:]

    @pl.when((i >= ATT1) & (i < QKV2))
    def _():
        b = i - ATT1
        bufB[pl.ds(K + b * QB, QB), :] = _att_compute(
            b, bufA, qkvS, wo1, bo1, aw1, ab1, f1a, f1ab, f1b, f1bb,
            g1w, g1b)

    @pl.when(i == QKV2)
    def _():
        qkvS[:, :] = _abt(bufB[K:K + T, :], wq2[:, :]) + bq2[:]

    @pl.when((i >= ATT2) & (i < CLS))
    def _():
        b = i - ATT2
        bufA[pl.ds(K + b * QB, QB), :] = _att_compute(
            b, bufB, qkvS, wo2, bo2, aw2, ab2, f2a, f2ab, f2b, f2bb,
            g2w, g2b)

    @pl.when(i == CLS)
    def _():
        hc = jnp.mean(bufA[K:K + T, :], axis=0, keepdims=True)
        h1 = jnp.maximum(_abt(hc, clw1[:, :]) + clb1[:], 0.0)
        logit_ref[:, :] = _abt(h1, clw2[:, :]) + clb2[:]

    @pl.when((i >= DEC1) & (i < DEC2))
    def _():
        b = i - DEC1
        win = bufA[pl.ds(b * QB, WB), :]
        _gat_store(b, win, bufB, w_d1, s_d1, d_d1, bd1, 4, 32, True)

    @pl.when((i >= DEC2) & (i < DEC3))
    def _():
        b = i - DEC2
        win = bufB[pl.ds(b * QB, WB), :]
        _gat_store(b, win, bufA, w_d2, s_d2, d_d2, bd2, 4, 32, True)

    @pl.when(i >= DEC3)
    def _():
        b = i - DEC3
        win = bufA[pl.ds(b * QB, WB), :]
        hhT = _abt(wt_d3[:, :], win)
        a_srcT = _abt(win, s_d3[:, :])
        a_dstT = _abt(d_d3[:, :], win[K:K + QB, :])
        jT = jax.lax.broadcasted_iota(jnp.int32, (WB, QB), 0)
        iqT = jax.lax.broadcasted_iota(jnp.int32, (WB, QB), 1)
        g = b * QB + jT - K
        band = (jT - iqT >= 0) & (jT - iqT <= 2 * K) & (g >= 0) & (g < T)
        madd = jnp.where(band, 0.0, NEG)
        s = a_srcT + a_dstT
        e = jnp.maximum(s, 0.2 * s) + madd
        m = jnp.max(e, axis=0, keepdims=True)
        w = jnp.exp(e - m)
        z = jnp.sum(w, axis=0, keepdims=True)
        oT = jnp.dot(hhT, w, preferred_element_type=jnp.float32)
        out_ref[:, :] = oT * (1.0 / (z + 1e-16)) + bcol_d3[:, :]


def kernel(window, params):
    x = window
    in_ch = x.shape[1]

    def _fold(p, heads, outd):
        w3 = p["W"].reshape(p["W"].shape[0], heads, outd)
        wsrc = jnp.einsum("iho,ho->hi", w3, p["att_src"])
        wdst = jnp.einsum("iho,ho->ih", w3, p["att_dst"])
        return wsrc, wdst

    gats = []
    for p in params["enc"] + params["dec"][:-1]:
        wsrc, wdst = _fold(p, 4, 32)
        gats += [p["W"], wsrc, wdst, p["b"]]
    t_ops = []
    for p in params["trans"]:
        t_ops += [p["Wqkv"], p["bqkv"], p["Wo"], p["bo"],
                  p["ln1_w"], p["ln1_b"], p["W1"], p["b1"],
                  p["W2"], p["b2"], p["ln2_w"], p["ln2_b"]]
    c = params["cls"]
    cls_ops = [c["W1"], c["b1"], c["W2"], c["b2"]]
    pd3 = params["dec"][-1]
    wsrc3, wdst3 = _fold(pd3, 1, 64)
    d3_ops = [pd3["W"].T, wsrc3, wdst3.T, pd3["b"][:, None]]

    operands = ([x, x, x] + gats[:12] + t_ops + cls_ops + gats[12:]
                + d3_ops)

    def _full(a):
        nd = a.ndim
        return pl.BlockSpec(a.shape, lambda i, _nd=nd: (0,) * _nd)

    in_specs = [
        pl.BlockSpec((QB, in_ch),
                     lambda i: (jnp.maximum(jnp.clip(i, 0, NBLK - 1) - 1, 0),
                                0)),
        pl.BlockSpec((QB, in_ch), lambda i: (jnp.clip(i, 0, NBLK - 1), 0)),
        pl.BlockSpec((QB, in_ch),
                     lambda i: (jnp.minimum(jnp.clip(i, 0, NBLK - 1) + 1,
                                            NBLK - 1), 0)),
    ] + [_full(a) for a in operands[3:]]

    out, logits = pl.pallas_call(
        _mega_body,
        grid=(NSTEP,),
        in_specs=in_specs,
        out_specs=[
            pl.BlockSpec((64, QB), lambda i: (0, jnp.clip(i - DEC3, 0,
                                                          NBLK - 1))),
            pl.BlockSpec((1, 2), lambda i: (0, 0)),
        ],
        out_shape=[
            jax.ShapeDtypeStruct((64, T), jnp.float32),
            jax.ShapeDtypeStruct((1, 2), jnp.float32),
        ],
        scratch_shapes=[
            pltpu.VMEM((T + 2 * K, HIDDEN), jnp.float32),
            pltpu.VMEM((T + 2 * K, HIDDEN), jnp.float32),
            pltpu.VMEM((T, 3 * HIDDEN), jnp.float32),
            pltpu.VMEM((WB, in_ch), jnp.float32),
        ],
    )(*operands)
    return (out, logits[0])
